# Initial kernel scaffold; baseline (speedup 1.0000x reference)
import functools

import jax
import jax.numpy as jnp
from jax import lax
from jax.experimental import pallas as pl
from jax.experimental.pallas import tpu as pltpu
from jax.experimental.pallas import tpu_sc as plsc

NUM_ROWS = 100000
DIM = 128
TOTAL = 1024 * 200

NC = 2
NS = 16
NW = NC * NS
PER_W = TOTAL // NW
CHUNK = 200
NCHUNK = PER_W // CHUNK


def _sc_gather(idx, weight):
    mesh = plsc.VectorSubcoreMesh(core_axis_name="c", subcore_axis_name="s")

    @functools.partial(
        pl.kernel,
        mesh=mesh,
        out_type=jax.ShapeDtypeStruct((TOTAL, DIM), jnp.float32),
        scratch_types=[
            pltpu.VMEM((NCHUNK, CHUNK), jnp.int32),
            pltpu.VMEM((CHUNK, DIM), jnp.float32),
            pltpu.VMEM((CHUNK, DIM), jnp.float32),
            pltpu.SemaphoreType.DMA,
            pltpu.SemaphoreType.DMA,
            pltpu.SemaphoreType.DMA,
            pltpu.SemaphoreType.DMA,
        ],
    )
    def k(table_hbm, idx_hbm, out_hbm, idx_v, rows0, rows1, g0, g1, o0, o1):
        wid = lax.axis_index("s") * NC + lax.axis_index("c")
        base = wid * PER_W
        pltpu.sync_copy(idx_hbm.at[wid], idx_v)

        @pl.loop(0, NCHUNK, step=2)
        def _(j):
            gw0 = pltpu.async_copy(table_hbm.at[idx_v.at[j]], rows0, g0)
            gw1 = pltpu.async_copy(table_hbm.at[idx_v.at[j + 1]], rows1, g1)
            gw0.wait()
            ow0 = pltpu.async_copy(
                rows0, out_hbm.at[pl.ds(base + j * CHUNK, CHUNK)], o0)
            gw1.wait()
            ow1 = pltpu.async_copy(
                rows1, out_hbm.at[pl.ds(base + (j + 1) * CHUNK, CHUNK)], o1)
            ow0.wait()
            ow1.wait()

    return k(weight, idx)


def kernel(token_ids, weight):
    idx = token_ids.astype(jnp.int32).reshape(NW, NCHUNK, CHUNK)
    out = _sc_gather(idx, weight.astype(jnp.float32))
    return out.reshape(token_ids.shape + (DIM,))


# speedup vs baseline: 7.1450x; 7.1450x over previous
import functools

import jax
import jax.numpy as jnp
from jax import lax
from jax.experimental import pallas as pl
from jax.experimental.pallas import tpu as pltpu
from jax.experimental.pallas import tpu_sc as plsc

NUM_ROWS = 100000
DIM = 128
TOTAL = 1024 * 200

NC = 2
NS = 16
NW = NC * NS
PER_W = TOTAL // NW
CHUNK = 128
NCHUNK = PER_W // CHUNK


def _sc_gather(idx, weight):
    mesh = plsc.VectorSubcoreMesh(core_axis_name="c", subcore_axis_name="s")

    @functools.partial(
        pl.kernel,
        mesh=mesh,
        out_type=jax.ShapeDtypeStruct((TOTAL, DIM), jnp.float32),
        scratch_types=[
            pltpu.VMEM((PER_W,), jnp.int32),
            pltpu.VMEM((CHUNK, DIM), jnp.float32),
            pltpu.VMEM((CHUNK, DIM), jnp.float32),
            pltpu.SemaphoreType.DMA,
            pltpu.SemaphoreType.DMA,
            pltpu.SemaphoreType.DMA,
            pltpu.SemaphoreType.DMA,
        ],
    )
    def k(table_hbm, idx_hbm, out_hbm, idx_v, rows0, rows1, g0, g1, o0, o1):
        wid = lax.axis_index("s") * NC + lax.axis_index("c")
        base = wid * PER_W
        pltpu.sync_copy(idx_hbm.at[wid], idx_v)

        @pl.loop(0, NCHUNK, step=2)
        def _(j):
            gw0 = pltpu.async_copy(
                table_hbm.at[idx_v.at[pl.ds(j * CHUNK, CHUNK)]], rows0, g0)
            gw1 = pltpu.async_copy(
                table_hbm.at[idx_v.at[pl.ds((j + 1) * CHUNK, CHUNK)]], rows1, g1)
            gw0.wait()
            ow0 = pltpu.async_copy(
                rows0, out_hbm.at[pl.ds(base + j * CHUNK, CHUNK)], o0)
            gw1.wait()
            ow1 = pltpu.async_copy(
                rows1, out_hbm.at[pl.ds(base + (j + 1) * CHUNK, CHUNK)], o1)
            ow0.wait()
            ow1.wait()

    return k(weight, idx)


def kernel(token_ids, weight):
    idx = token_ids.astype(jnp.int32).reshape(NW, PER_W)
    out = _sc_gather(idx, weight.astype(jnp.float32))
    return out.reshape(token_ids.shape + (DIM,))


# speedup vs baseline: 7.1666x; 1.0030x over previous
import functools

import jax
import jax.numpy as jnp
from jax import lax
from jax.experimental import pallas as pl
from jax.experimental.pallas import tpu as pltpu
from jax.experimental.pallas import tpu_sc as plsc

NUM_ROWS = 100000
DIM = 128
TOTAL = 1024 * 200

NC = 2
NS = 16
NW = NC * NS
PER_W = TOTAL // NW
CHUNK = 128
NCHUNK = PER_W // CHUNK


def _sc_gather(idx, weight):
    mesh = plsc.VectorSubcoreMesh(core_axis_name="c", subcore_axis_name="s")

    @functools.partial(
        pl.kernel,
        mesh=mesh,
        out_type=jax.ShapeDtypeStruct((TOTAL, DIM), jnp.float32),
        scratch_types=[
            pltpu.VMEM((PER_W,), jnp.int32),
            pltpu.VMEM((CHUNK, DIM), jnp.float32),
            pltpu.VMEM((CHUNK, DIM), jnp.float32),
            pltpu.SemaphoreType.DMA,
            pltpu.SemaphoreType.DMA,
            pltpu.SemaphoreType.DMA,
            pltpu.SemaphoreType.DMA,
        ],
    )
    def k(table_hbm, idx_hbm, out_hbm, idx_v, rows0, rows1, g0, g1, o0, o1):
        wid = lax.axis_index("s") * NC + lax.axis_index("c")
        base = wid * PER_W
        pltpu.sync_copy(idx_hbm.at[wid], idx_v)

        @pl.loop(0, NCHUNK, step=2)
        def _(j):
            @pl.when(j > 0)
            def _():
                pltpu.make_async_copy(
                    rows0, out_hbm.at[pl.ds(base + (j - 2) * CHUNK, CHUNK)], o0
                ).wait()
                pltpu.make_async_copy(
                    rows1, out_hbm.at[pl.ds(base + (j - 1) * CHUNK, CHUNK)], o1
                ).wait()

            gw0 = pltpu.async_copy(
                table_hbm.at[idx_v.at[pl.ds(j * CHUNK, CHUNK)]], rows0, g0)
            gw1 = pltpu.async_copy(
                table_hbm.at[idx_v.at[pl.ds((j + 1) * CHUNK, CHUNK)]], rows1, g1)
            gw0.wait()
            pltpu.async_copy(
                rows0, out_hbm.at[pl.ds(base + j * CHUNK, CHUNK)], o0)
            gw1.wait()
            pltpu.async_copy(
                rows1, out_hbm.at[pl.ds(base + (j + 1) * CHUNK, CHUNK)], o1)

        pltpu.make_async_copy(
            rows0, out_hbm.at[pl.ds(base + (NCHUNK - 2) * CHUNK, CHUNK)], o0
        ).wait()
        pltpu.make_async_copy(
            rows1, out_hbm.at[pl.ds(base + (NCHUNK - 1) * CHUNK, CHUNK)], o1
        ).wait()

    return k(weight, idx)


def kernel(token_ids, weight):
    idx = token_ids.astype(jnp.int32).reshape(NW, PER_W)
    out = _sc_gather(idx, weight.astype(jnp.float32))
    return out.reshape(token_ids.shape + (DIM,))


# speedup vs baseline: 7.5919x; 1.0593x over previous
import functools

import jax
import jax.numpy as jnp
from jax import lax
from jax.experimental import pallas as pl
from jax.experimental.pallas import tpu as pltpu
from jax.experimental.pallas import tpu_sc as plsc

NUM_ROWS = 100000
DIM = 128
TOTAL = 1024 * 200

NC = 2
NS = 16
NW = NC * NS
PER_W = TOTAL // NW
CHUNK = 128
NCHUNK = PER_W // CHUNK
NMAIN = (NCHUNK // 4) * 4


def _sc_gather(idx, weight):
    mesh = plsc.VectorSubcoreMesh(core_axis_name="c", subcore_axis_name="s")

    @functools.partial(
        pl.kernel,
        mesh=mesh,
        out_type=jax.ShapeDtypeStruct((TOTAL, DIM), jnp.float32),
        scratch_types=[
            pltpu.VMEM((PER_W,), jnp.int32),
            pltpu.VMEM((CHUNK, DIM), jnp.float32),
            pltpu.VMEM((CHUNK, DIM), jnp.float32),
            pltpu.VMEM((CHUNK, DIM), jnp.float32),
            pltpu.VMEM((CHUNK, DIM), jnp.float32),
            pltpu.SemaphoreType.DMA,
            pltpu.SemaphoreType.DMA,
            pltpu.SemaphoreType.DMA,
            pltpu.SemaphoreType.DMA,
            pltpu.SemaphoreType.DMA,
            pltpu.SemaphoreType.DMA,
            pltpu.SemaphoreType.DMA,
            pltpu.SemaphoreType.DMA,
        ],
    )
    def k(table_hbm, idx_hbm, out_hbm, idx_v,
          r0, r1, r2, r3, g0, g1, g2, g3, o0, o1, o2, o3):
        wid = lax.axis_index("s") * NC + lax.axis_index("c")
        base = wid * PER_W
        pltpu.sync_copy(idx_hbm.at[wid], idx_v)

        def oslice(j):
            return out_hbm.at[pl.ds(base + j * CHUNK, CHUNK)]

        def gather(j, buf, gsem):
            return pltpu.async_copy(
                table_hbm.at[idx_v.at[pl.ds(j * CHUNK, CHUNK)]], buf, gsem)

        bufs = (r0, r1, r2, r3)
        gsems = (g0, g1, g2, g3)
        osems = (o0, o1, o2, o3)

        @pl.loop(0, NMAIN, step=4)
        def _(j):
            @pl.when(j > 0)
            def _():
                for b in range(4):
                    pltpu.make_async_copy(
                        bufs[b], oslice(j - 4 + b), osems[b]).wait()

            gws = [gather(j + b, bufs[b], gsems[b]) for b in range(4)]
            for b in range(4):
                gws[b].wait()
                pltpu.async_copy(bufs[b], oslice(j + b), osems[b])

        for b in range(4):
            pltpu.make_async_copy(
                bufs[b], oslice(NMAIN - 4 + b), osems[b]).wait()

        gwa = gather(NMAIN, r0, g0)
        gwb = gather(NMAIN + 1, r1, g1)
        gwa.wait()
        pltpu.async_copy(r0, oslice(NMAIN), o0)
        gwb.wait()
        pltpu.async_copy(r1, oslice(NMAIN + 1), o1)
        pltpu.make_async_copy(r0, oslice(NMAIN), o0).wait()
        pltpu.make_async_copy(r1, oslice(NMAIN + 1), o1).wait()

    return k(weight, idx)


def kernel(token_ids, weight):
    idx = token_ids.astype(jnp.int32).reshape(NW, PER_W)
    out = _sc_gather(idx, weight.astype(jnp.float32))
    return out.reshape(token_ids.shape + (DIM,))


# speedup vs baseline: 7.9128x; 1.0423x over previous
import functools

import jax
import jax.numpy as jnp
from jax import lax
from jax.experimental import pallas as pl
from jax.experimental.pallas import tpu as pltpu
from jax.experimental.pallas import tpu_sc as plsc

NUM_ROWS = 100000
DIM = 128
TOTAL = 1024 * 200

NC = 2
NS = 16
NW = NC * NS
PER_W = TOTAL // NW
CHUNK = 128
NCHUNK = PER_W // CHUNK
NB = 6
NMAIN = (NCHUNK // NB) * NB


def _sc_gather(idx, weight):
    mesh = plsc.VectorSubcoreMesh(core_axis_name="c", subcore_axis_name="s")

    @functools.partial(
        pl.kernel,
        mesh=mesh,
        out_type=jax.ShapeDtypeStruct((TOTAL, DIM), jnp.float32),
        scratch_types=[
            pltpu.VMEM((PER_W,), jnp.int32),
        ] + [pltpu.VMEM((CHUNK, DIM), jnp.float32)] * NB
          + [pltpu.SemaphoreType.DMA] * (2 * NB),
    )
    def k(table_hbm, idx_hbm, out_hbm, idx_v, *bufs_and_sems):
        bufs = bufs_and_sems[:NB]
        gsems = bufs_and_sems[NB:2 * NB]
        osems = bufs_and_sems[2 * NB:3 * NB]
        wid = lax.axis_index("s") * NC + lax.axis_index("c")
        base = wid * PER_W
        pltpu.sync_copy(idx_hbm.at[wid], idx_v)

        def oslice(j):
            return out_hbm.at[pl.ds(base + j * CHUNK, CHUNK)]

        def gather(j, buf, gsem):
            return pltpu.async_copy(
                table_hbm.at[idx_v.at[pl.ds(j * CHUNK, CHUNK)]], buf, gsem)

        @pl.loop(0, NMAIN, step=NB)
        def _(j):
            for b in range(NB):
                @pl.when(j > 0)
                def _(b=b):
                    pltpu.make_async_copy(
                        bufs[b], oslice(j - NB + b), osems[b]).wait()
                gather(j + b, bufs[b], gsems[b])
            for b in range(NB):
                pltpu.make_async_copy(
                    table_hbm.at[idx_v.at[pl.ds((j + b) * CHUNK, CHUNK)]],
                    bufs[b], gsems[b]).wait()
                pltpu.async_copy(bufs[b], oslice(j + b), osems[b])

        for b in range(NB):
            pltpu.make_async_copy(
                bufs[b], oslice(NMAIN - NB + b), osems[b]).wait()

        for b in range(NCHUNK - NMAIN):
            gather(NMAIN + b, bufs[b], gsems[b])
        for b in range(NCHUNK - NMAIN):
            pltpu.make_async_copy(
                table_hbm.at[idx_v.at[pl.ds((NMAIN + b) * CHUNK, CHUNK)]],
                bufs[b], gsems[b]).wait()
            pltpu.async_copy(bufs[b], oslice(NMAIN + b), osems[b])
        for b in range(NCHUNK - NMAIN):
            pltpu.make_async_copy(
                bufs[b], oslice(NMAIN + b), osems[b]).wait()

    return k(weight, idx)


def kernel(token_ids, weight):
    idx = token_ids.astype(jnp.int32).reshape(NW, PER_W)
    out = _sc_gather(idx, weight.astype(jnp.float32))
    return out.reshape(token_ids.shape + (DIM,))
